# baseline (speedup 1.0000x reference)
import functools

import jax
import jax.numpy as jnp
from jax import lax
from jax.experimental import pallas as pl
from jax.experimental.pallas import tpu as pltpu
from jax.experimental.pallas import tpu_sc as plsc

_SIGMA = 0.5
_LANES = 16
_NWORKERS = 32
_RSPLIT = 1024
_BI = 256
_BJ = 512


def _sc_body(npad, gpw, x1h, y1h, x2h, y2h, sh, outh,
             x1v, y1v, x2v, y2v, sv, av, resv, sem):
    copies = [pltpu.make_async_copy(x1h, x1v, sem),
              pltpu.make_async_copy(y1h, y1v, sem),
              pltpu.make_async_copy(x2h, x2v, sem),
              pltpu.make_async_copy(y2h, y2v, sem),
              pltpu.make_async_copy(sh, sv, sem)]
    for c in copies:
        c.start()
    for c in copies:
        c.wait()

    def area_body(t, carry):
        sl = pl.ds(t * _LANES, _LANES)
        av[sl] = (x2v[sl] - x1v[sl]) * (y2v[sl] - y1v[sl])
        return carry

    lax.fori_loop(0, npad // _LANES, area_body, 0)

    wid = lax.axis_index("s") * 2 + lax.axis_index("c")

    def group_body(gi, carry):
        g = wid * gpw + gi
        base = g * _LANES
        sl = pl.ds(base, _LANES)
        xi1 = x1v[sl]
        yi1 = y1v[sl]
        xi2 = x2v[sl]
        yi2 = y2v[sl]
        si = sv[sl]
        ai = av[sl]

        def j_body(jb, accs):
            accs = list(accs)
            jsl = pl.ds(jb * _LANES, _LANES)
            xj1 = x1v[jsl]
            yj1 = y1v[jsl]
            xj2 = x2v[jsl]
            yj2 = y2v[jsl]
            sj = sv[jsl]
            aj = av[jsl]
            for l in range(_LANES):
                xx1 = jnp.maximum(xi1, xj1[l])
                yy1 = jnp.maximum(yi1, yj1[l])
                xx2 = jnp.minimum(xi2, xj2[l])
                yy2 = jnp.minimum(yi2, yj2[l])
                w = jnp.maximum(xx2 - xx1, 0.0)
                h = jnp.maximum(yy2 - yy1, 0.0)
                inter = w * h
                union = jnp.maximum(ai + aj[l] - inter, 1e-8)
                iou = inter / union
                k = l % 4
                accs[k] = accs[k] + jnp.where(sj[l] > si, iou * iou, 0.0)
            return tuple(accs)

        zero = jnp.zeros((_LANES,), jnp.float32)
        accs = lax.fori_loop(0, npad // _LANES, j_body,
                             (zero, zero, zero, zero))
        acc = (accs[0] + accs[1]) + (accs[2] + accs[3])
        resv[...] = si * jnp.exp(acc * (-1.0 / _SIGMA))
        pltpu.sync_copy(resv, outh.at[pl.ds(base, _LANES)])
        return carry

    lax.fori_loop(0, gpw, group_body, 0)


def _tc_body(npad, cols_ref, rows_ref, out_ref):
    cols = cols_ref[...]
    xi1 = cols[:, 0:1]
    yi1 = cols[:, 1:2]
    xi2 = cols[:, 2:3]
    yi2 = cols[:, 3:4]
    si = cols[:, 4:5]
    ai = (xi2 - xi1) * (yi2 - yi1)

    def j_body(jb, acc):
        jsl = pl.ds(jb * _BJ, _BJ)
        xj1 = rows_ref[0:1, jsl]
        yj1 = rows_ref[1:2, jsl]
        xj2 = rows_ref[2:3, jsl]
        yj2 = rows_ref[3:4, jsl]
        sj = rows_ref[4:5, jsl]
        aj = (xj2 - xj1) * (yj2 - yj1)
        xx1 = jnp.maximum(xi1, xj1)
        yy1 = jnp.maximum(yi1, yj1)
        xx2 = jnp.minimum(xi2, xj2)
        yy2 = jnp.minimum(yi2, yj2)
        w = jnp.maximum(xx2 - xx1, 0.0)
        h = jnp.maximum(yy2 - yy1, 0.0)
        inter = w * h
        union = jnp.maximum(ai + aj - inter, 1e-8)
        iou = inter / union
        contrib = jnp.where(sj > si, iou * iou, 0.0)
        return acc + jnp.sum(contrib, axis=1, keepdims=True)

    acc = lax.fori_loop(0, npad // _BJ, j_body,
                        jnp.zeros((_BI, 1), jnp.float32), unroll=2)
    out_ref[...] = si * jnp.exp(acc * (-1.0 / _SIGMA))


@jax.jit
def kernel(boxes, scores):
    n = boxes.shape[0]
    chunk = _LANES * _NWORKERS
    npad = ((n + chunk - 1) // chunk) * chunk

    x1 = jnp.zeros((npad,), jnp.float32).at[:n].set(boxes[:, 0])
    y1 = jnp.zeros((npad,), jnp.float32).at[:n].set(boxes[:, 1])
    x2 = jnp.zeros((npad,), jnp.float32).at[:n].set(boxes[:, 2])
    y2 = jnp.zeros((npad,), jnp.float32).at[:n].set(boxes[:, 3])
    s = jnp.full((npad,), -1.0, jnp.float32).at[:n].set(scores)

    gpw = _RSPLIT // (_LANES * _NWORKERS)
    mesh = plsc.VectorSubcoreMesh(core_axis_name="c", subcore_axis_name="s")
    sc_out = pl.kernel(
        functools.partial(_sc_body, npad, gpw),
        out_type=jax.ShapeDtypeStruct((_RSPLIT,), jnp.float32),
        mesh=mesh,
        scratch_types=[pltpu.VMEM((npad,), jnp.float32)] * 6
        + [pltpu.VMEM((_LANES,), jnp.float32), pltpu.SemaphoreType.DMA],
    )(x1, y1, x2, y2, s)

    cols = jnp.stack([x1, y1, x2, y2, s, s, s, s], axis=1)
    rows = jnp.stack([x1, y1, x2, y2, s, s, s, s], axis=0)
    ntc = npad - _RSPLIT
    tc_out = pl.pallas_call(
        functools.partial(_tc_body, npad),
        grid=(ntc // _BI,),
        in_specs=[
            pl.BlockSpec((_BI, 8), lambda i: (i + _RSPLIT // _BI, 0)),
            pl.BlockSpec((8, npad), lambda i: (0, 0)),
        ],
        out_specs=pl.BlockSpec((_BI, 1), lambda i: (i, 0)),
        out_shape=jax.ShapeDtypeStruct((ntc, 1), jnp.float32),
    )(cols, rows)

    return jnp.concatenate([sc_out, tc_out[:, 0]])[:n]


# speedup vs baseline: 1.0533x; 1.0533x over previous
import functools

import jax
import jax.numpy as jnp
from jax import lax
from jax.experimental import pallas as pl
from jax.experimental.pallas import tpu as pltpu
from jax.experimental.pallas import tpu_sc as plsc

_SIGMA = 0.5
_LANES = 16
_NWORKERS = 32
_RSPLIT = 1024
_BI = 256
_BJ = 512


def _sc_body(npad, gpw, rows_h, outh,
             x1v, y1v, x2v, y2v, sv, av, resv, sem):
    copies = [pltpu.make_async_copy(rows_h.at[0], x1v, sem),
              pltpu.make_async_copy(rows_h.at[1], y1v, sem),
              pltpu.make_async_copy(rows_h.at[2], x2v, sem),
              pltpu.make_async_copy(rows_h.at[3], y2v, sem),
              pltpu.make_async_copy(rows_h.at[4], sv, sem)]
    for c in copies:
        c.start()
    for c in copies:
        c.wait()

    def area_body(t, carry):
        sl = pl.ds(t * _LANES, _LANES)
        av[sl] = (x2v[sl] - x1v[sl]) * (y2v[sl] - y1v[sl])
        return carry

    lax.fori_loop(0, npad // _LANES, area_body, 0)

    wid = lax.axis_index("s") * 2 + lax.axis_index("c")

    def group_body(gi, carry):
        g = wid * gpw + gi
        base = g * _LANES
        sl = pl.ds(base, _LANES)
        xi1 = x1v[sl]
        yi1 = y1v[sl]
        xi2 = x2v[sl]
        yi2 = y2v[sl]
        si = sv[sl]
        ai = av[sl]

        def j_body(jb, accs):
            accs = list(accs)
            jsl = pl.ds(jb * _LANES, _LANES)
            xj1 = x1v[jsl]
            yj1 = y1v[jsl]
            xj2 = x2v[jsl]
            yj2 = y2v[jsl]
            sj = sv[jsl]
            aj = av[jsl]
            for l in range(_LANES):
                xx1 = jnp.maximum(xi1, xj1[l])
                yy1 = jnp.maximum(yi1, yj1[l])
                xx2 = jnp.minimum(xi2, xj2[l])
                yy2 = jnp.minimum(yi2, yj2[l])
                w = jnp.maximum(xx2 - xx1, 0.0)
                h = jnp.maximum(yy2 - yy1, 0.0)
                inter = w * h
                union = ai + aj[l] - inter
                iou = inter / union
                k = l % 4
                accs[k] = accs[k] + jnp.where(sj[l] > si, iou * iou, 0.0)
            return tuple(accs)

        zero = jnp.zeros((_LANES,), jnp.float32)
        accs = lax.fori_loop(0, npad // _LANES, j_body,
                             (zero, zero, zero, zero))
        acc = (accs[0] + accs[1]) + (accs[2] + accs[3])
        resv[...] = si * jnp.exp(acc * (-1.0 / _SIGMA))
        pltpu.sync_copy(resv, outh.at[pl.ds(base, _LANES)])
        return carry

    lax.fori_loop(0, gpw, group_body, 0)


def _tc_body(npad, cols_ref, rows_ref, out_ref):
    cols = cols_ref[...]
    xi1 = cols[:, 0:1]
    yi1 = cols[:, 1:2]
    xi2 = cols[:, 2:3]
    yi2 = cols[:, 3:4]
    si = cols[:, 4:5]
    ai = (xi2 - xi1) * (yi2 - yi1)

    def j_body(jb, acc):
        jsl = pl.ds(jb * _BJ, _BJ)
        xj1 = rows_ref[0:1, jsl]
        yj1 = rows_ref[1:2, jsl]
        xj2 = rows_ref[2:3, jsl]
        yj2 = rows_ref[3:4, jsl]
        sj = rows_ref[4:5, jsl]
        aj = (xj2 - xj1) * (yj2 - yj1)
        xx1 = jnp.maximum(xi1, xj1)
        yy1 = jnp.maximum(yi1, yj1)
        xx2 = jnp.minimum(xi2, xj2)
        yy2 = jnp.minimum(yi2, yj2)
        w = jnp.maximum(xx2 - xx1, 0.0)
        h = jnp.maximum(yy2 - yy1, 0.0)
        inter = w * h
        union = ai + aj - inter
        iou = inter / union
        contrib = jnp.where(sj > si, iou * iou, 0.0)
        return acc + jnp.sum(contrib, axis=1, keepdims=True)

    acc = lax.fori_loop(0, npad // _BJ, j_body,
                        jnp.zeros((_BI, 1), jnp.float32), unroll=2)
    out_ref[...] = si * jnp.exp(acc * (-1.0 / _SIGMA))


@jax.jit
def kernel(boxes, scores):
    n = boxes.shape[0]
    chunk = _LANES * _NWORKERS
    npad = ((n + chunk - 1) // chunk) * chunk

    s = jnp.full((npad,), -1.0, jnp.float32).at[:n].set(scores)
    cols = jnp.zeros((npad, 8), jnp.float32).at[:n, 0:4].set(boxes)
    cols = cols.at[:, 4].set(s)
    rows = cols.T

    gpw = _RSPLIT // (_LANES * _NWORKERS)
    mesh = plsc.VectorSubcoreMesh(core_axis_name="c", subcore_axis_name="s")
    sc_out = pl.kernel(
        functools.partial(_sc_body, npad, gpw),
        out_type=jax.ShapeDtypeStruct((_RSPLIT,), jnp.float32),
        mesh=mesh,
        scratch_types=[pltpu.VMEM((npad,), jnp.float32)] * 6
        + [pltpu.VMEM((_LANES,), jnp.float32), pltpu.SemaphoreType.DMA],
    )(rows)

    ntc = npad - _RSPLIT
    tc_out = pl.pallas_call(
        functools.partial(_tc_body, npad),
        grid=(ntc // _BI,),
        in_specs=[
            pl.BlockSpec((_BI, 8), lambda i: (i + _RSPLIT // _BI, 0)),
            pl.BlockSpec((8, npad), lambda i: (0, 0)),
        ],
        out_specs=pl.BlockSpec((_BI, 1), lambda i: (i, 0)),
        out_shape=jax.ShapeDtypeStruct((ntc, 1), jnp.float32),
    )(cols, rows)

    return jnp.concatenate([sc_out, tc_out[:, 0]])[:n]


# speedup vs baseline: 1.1987x; 1.1380x over previous
import functools

import jax
import jax.numpy as jnp
from jax import lax
from jax.experimental import pallas as pl
from jax.experimental.pallas import tpu as pltpu
from jax.experimental.pallas import tpu_sc as plsc

_SIGMA = 0.5
_LANES = 16
_NWORKERS = 32
_RSPLIT = 512
_BI = 512
_BJ = 512


def _sc_body(npad, gpw, rows_h, outh,
             x1v, y1v, x2v, y2v, sv, av, resv, sem):
    copies = [pltpu.make_async_copy(rows_h.at[0], x1v, sem),
              pltpu.make_async_copy(rows_h.at[1], y1v, sem),
              pltpu.make_async_copy(rows_h.at[2], x2v, sem),
              pltpu.make_async_copy(rows_h.at[3], y2v, sem),
              pltpu.make_async_copy(rows_h.at[4], sv, sem)]
    for c in copies:
        c.start()
    for c in copies:
        c.wait()

    def area_body(t, carry):
        sl = pl.ds(t * _LANES, _LANES)
        av[sl] = (x2v[sl] - x1v[sl]) * (y2v[sl] - y1v[sl])
        return carry

    lax.fori_loop(0, npad // _LANES, area_body, 0)

    wid = lax.axis_index("s") * 2 + lax.axis_index("c")

    def group_body(gi, carry):
        g = wid * gpw + gi
        base = g * _LANES
        sl = pl.ds(base, _LANES)
        xi1 = x1v[sl]
        yi1 = y1v[sl]
        xi2 = x2v[sl]
        yi2 = y2v[sl]
        si = sv[sl]
        ai = av[sl]

        def j_body(jb, accs):
            accs = list(accs)
            jsl = pl.ds(jb * _LANES, _LANES)
            xj1 = x1v[jsl]
            yj1 = y1v[jsl]
            xj2 = x2v[jsl]
            yj2 = y2v[jsl]
            sj = sv[jsl]
            aj = av[jsl]
            for l in range(_LANES):
                xx1 = jnp.maximum(xi1, xj1[l])
                yy1 = jnp.maximum(yi1, yj1[l])
                xx2 = jnp.minimum(xi2, xj2[l])
                yy2 = jnp.minimum(yi2, yj2[l])
                w = jnp.maximum(xx2 - xx1, 0.0)
                h = jnp.maximum(yy2 - yy1, 0.0)
                inter = w * h
                union = ai + aj[l] - inter
                iou = inter / union
                k = l % 4
                accs[k] = accs[k] + jnp.where(sj[l] > si, iou * iou, 0.0)
            return tuple(accs)

        zero = jnp.zeros((_LANES,), jnp.float32)
        accs = lax.fori_loop(0, npad // _LANES, j_body,
                             (zero, zero, zero, zero))
        acc = (accs[0] + accs[1]) + (accs[2] + accs[3])
        resv[...] = si * jnp.exp(acc * (-1.0 / _SIGMA))
        pltpu.sync_copy(resv, outh.at[pl.ds(base, _LANES)])
        return carry

    lax.fori_loop(0, gpw, group_body, 0)


def _tc_body(npad, cols_ref, rows_ref, out_ref):
    cols = cols_ref[...]
    xi1 = cols[:, 0:1]
    yi1 = cols[:, 1:2]
    xi2 = cols[:, 2:3]
    yi2 = cols[:, 3:4]
    si = cols[:, 4:5]
    ai = (xi2 - xi1) * (yi2 - yi1)

    def j_body(jb, acc):
        jsl = pl.ds(jb * _BJ, _BJ)
        xj1 = rows_ref[0:1, jsl]
        yj1 = rows_ref[1:2, jsl]
        xj2 = rows_ref[2:3, jsl]
        yj2 = rows_ref[3:4, jsl]
        sj = rows_ref[4:5, jsl]
        aj = (xj2 - xj1) * (yj2 - yj1)
        xx1 = jnp.maximum(xi1, xj1)
        yy1 = jnp.maximum(yi1, yj1)
        xx2 = jnp.minimum(xi2, xj2)
        yy2 = jnp.minimum(yi2, yj2)
        w = jnp.maximum(xx2 - xx1, 0.0)
        h = jnp.maximum(yy2 - yy1, 0.0)
        inter = w * h
        union = ai + aj - inter
        iou = inter / union
        contrib = jnp.where(sj > si, iou * iou, 0.0)
        return acc + jnp.sum(contrib, axis=1, keepdims=True)

    acc = lax.fori_loop(0, npad // _BJ, j_body,
                        jnp.zeros((_BI, 1), jnp.float32), unroll=10)
    out_ref[...] = si * jnp.exp(acc * (-1.0 / _SIGMA))


@jax.jit
def kernel(boxes, scores):
    n = boxes.shape[0]
    chunk = _LANES * _NWORKERS
    npad = ((n + chunk - 1) // chunk) * chunk

    s = jnp.full((npad,), -1.0, jnp.float32).at[:n].set(scores)
    cols = jnp.zeros((npad, 8), jnp.float32).at[:n, 0:4].set(boxes)
    cols = cols.at[:, 4].set(s)
    rows = cols.T

    gpw = _RSPLIT // (_LANES * _NWORKERS)
    mesh = plsc.VectorSubcoreMesh(core_axis_name="c", subcore_axis_name="s")
    sc_out = pl.kernel(
        functools.partial(_sc_body, npad, gpw),
        out_type=jax.ShapeDtypeStruct((_RSPLIT,), jnp.float32),
        mesh=mesh,
        scratch_types=[pltpu.VMEM((npad,), jnp.float32)] * 6
        + [pltpu.VMEM((_LANES,), jnp.float32), pltpu.SemaphoreType.DMA],
    )(rows)

    ntc = npad - _RSPLIT
    tc_out = pl.pallas_call(
        functools.partial(_tc_body, npad),
        grid=(ntc // _BI,),
        in_specs=[
            pl.BlockSpec((_BI, 8), lambda i: (i + _RSPLIT // _BI, 0)),
            pl.BlockSpec((8, npad), lambda i: (0, 0)),
        ],
        out_specs=pl.BlockSpec((_BI, 1), lambda i: (i, 0)),
        out_shape=jax.ShapeDtypeStruct((ntc, 1), jnp.float32),
    )(cols, rows)

    return jnp.concatenate([sc_out, tc_out[:, 0]])[:n]
